# baseline (speedup 1.0000x reference)
import functools

import jax
import jax.numpy as jnp
from jax import lax
from jax.experimental import pallas as pl
from jax.experimental.pallas import tpu as pltpu
from jax.experimental.pallas import tpu_sc as plsc

HIST = 200
BATCH = 16384
D = 16
VOCAB = 1000000
NC = 2
NS = 16
NW = NC * NS
CHUNK = 1024
HG = HIST // 8
BT = BATCH // 128
NCELL = HG * BT
CPW = NCELL // NW
VT = 7936
VPAD = VT * 128
K0 = 4
NJ0 = VT // (NW * K0)
MAGIC = 123456.0


@functools.lru_cache(maxsize=None)
def _build():
    mesh = plsc.VectorSubcoreMesh(core_axis_name="c", subcore_axis_name="s")

    @functools.partial(
        pl.kernel,
        out_type=(
            jax.ShapeDtypeStruct((HIST, D // 8, BT, 8, 128), jnp.float32),
            jax.ShapeDtypeStruct((NC, VPAD, D), jnp.float32),
            jax.ShapeDtypeStruct((NC, 16), jnp.float32),
        ),
        mesh=mesh,
        scratch_types=[
            pltpu.VMEM((2, CHUNK), jnp.int32),
            pltpu.VMEM((2, CHUNK, D), jnp.float32),
            pltpu.VMEM((2, 8, D // 8, 8, 128), jnp.float32),
            pltpu.VMEM((2, D // 8, K0, 8, 128), jnp.float32),
            pltpu.VMEM((2, K0 * 128, D), jnp.float32),
            pltpu.VMEM((16,), jnp.float32),
            pltpu.VMEM((16,), jnp.float32),
            pltpu.SemaphoreType.DMA,
            pltpu.SemaphoreType.DMA,
            pltpu.SemaphoreType.DMA,
            pltpu.SemaphoreType.DMA,
            pltpu.SemaphoreType.DMA,
            pltpu.SemaphoreType.DMA,
            pltpu.SemaphoreType.DMA,
            pltpu.SemaphoreType.DMA,
            pltpu.SemaphoreType.DMA,
            pltpu.SemaphoreType.DMA,
            pltpu.SemaphoreType.DMA,
            pltpu.SemaphoreType.DMA,
        ],
        compiler_params=pltpu.CompilerParams(
            use_tc_tiling_on_sc=False, needs_layout_passes=False
        ),
    )
    def gather_kernel(idx_hbm, tab4_hbm, out_hbm, tab_hbm, flags_hbm, idx_v,
                      rows_v, out_v, tin_v, trows_v, flag_v, pflag_v, sidx0,
                      sidx1, sgat0, sgat1, sout0, sout1, stin0, stin1,
                      stout0, stout1, sflag, spoll):
        cid = lax.axis_index("c")
        sid = lax.axis_index("s")
        wid = sid * NC + cid
        base = wid * CPW
        lane = lax.iota(jnp.int32, 16)

        table = tab_hbm.at[cid]

        @pl.when(sid == 0)
        def _():
            flag_v[...] = jnp.zeros((16,), jnp.float32)
            pltpu.async_copy(flag_v, flags_hbm.at[cid], sflag).wait()

        def idx_slice(cell):
            return idx_hbm.at[cell, :]

        def issue_idx(cell, b, sem):
            pltpu.async_copy(idx_slice(cell), idx_v.at[b], sem)

        def wait_idx(b, sem):
            pltpu.make_async_copy(idx_slice(0), idx_v.at[b], sem).wait()

        lo0 = wid * (NJ0 * K0)

        def tin_slice(j):
            return tab4_hbm.at[:, pl.ds(lo0 + j * K0, K0), :, :]

        _stin = (stin0, stin1)
        _stout = (stout0, stout1)

        def issue_tin(j, b):
            pltpu.async_copy(tin_slice(j), tin_v.at[b], _stin[b])

        def wait_tin(b):
            pltpu.make_async_copy(tin_slice(0), tin_v.at[b], _stin[b]).wait()

        def trows_slice(j, c):
            return tab_hbm.at[c, pl.ds((lo0 + j * K0) * 128, K0 * 128), :]

        def issue_tout(j, b):
            pltpu.async_copy(trows_v.at[b], trows_slice(j, 0), _stout[b])
            pltpu.async_copy(trows_v.at[b], trows_slice(j, 1), _stout[b])

        def wait_tout(b):
            pltpu.make_async_copy(trows_v.at[b], trows_slice(0, 0),
                                  _stout[b]).wait()
            pltpu.make_async_copy(trows_v.at[b], trows_slice(0, 1),
                                  _stout[b]).wait()

        def transpose_vt(b):
            @plsc.parallel_loop(0, K0 * 8)
            def _vl_loop(i):
                k = i >> 3
                v16 = (i & 7) * 16
                rowv = k * 128 + v16 + lane
                for dg in range(D // 8):
                    for ds in range(8):
                        col = jnp.full((16,), dg * 8 + ds, jnp.int32)
                        vals = tin_v[b, dg, k, ds, pl.ds(v16, 16)]
                        plsc.store_scatter(trows_v.at[b], [rowv, col], vals)

        issue_idx(base, 0, sidx0)
        issue_idx(base + 1, 1, sidx1)

        issue_tin(0, 0)

        @pl.loop(0, NJ0, step=2)
        def _t_loop(j):
            issue_tin(j + 1, 1)
            wait_tin(0)

            @pl.when(j >= 2)
            def _():
                wait_tout(0)

            transpose_vt(0)
            issue_tout(j, 0)

            @pl.when(j + 2 < NJ0)
            def _():
                issue_tin(j + 2, 0)

            wait_tin(1)

            @pl.when(j >= 2)
            def _():
                wait_tout(1)

            transpose_vt(1)
            issue_tout(j + 1, 1)

        wait_tout(0)
        wait_tout(1)
        plsc.subcore_barrier()

        @pl.when(sid == 0)
        def _():
            flag_v[...] = jnp.full((16,), MAGIC, jnp.float32)
            pltpu.async_copy(flag_v, flags_hbm.at[cid], sflag).wait()

            def _poll_body(carry):
                pltpu.async_copy(flags_hbm.at[1 - cid], pflag_v, spoll).wait()
                hits = (pflag_v[...] == MAGIC).astype(jnp.int32)
                return jnp.sum(hits) != 16

            lax.while_loop(lambda c: c, _poll_body, jnp.bool_(True))

        plsc.subcore_barrier()

        def out_slice(cell):
            return out_hbm.at[pl.ds((cell // BT) * 8, 8), :, cell % BT, :, :]

        def issue_gather(b, sem):
            pltpu.async_copy(table.at[idx_v.at[b]], rows_v.at[b], sem)

        def wait_gather(b, sem):
            pltpu.make_async_copy(table.at[idx_v.at[b]], rows_v.at[b],
                                  sem).wait()

        def issue_out(cell, b, sem):
            pltpu.async_copy(out_v.at[b], out_slice(cell), sem)

        def wait_out(b, sem):
            pltpu.make_async_copy(out_v.at[b], out_slice(0), sem).wait()

        def transpose(b):
            @plsc.parallel_loop(0, 8)
            def _hs_loop(hs):
                row_base = hs * 128
                for dg in range(D // 8):
                    for ds in range(8):
                        col = jnp.full((16,), dg * 8 + ds, jnp.int32)
                        for j in range(8):
                            rows = row_base + j * 16 + lane
                            vals = plsc.load_gather(rows_v.at[b], [rows, col])
                            out_v[b, hs, dg, ds, pl.ds(j * 16, 16)] = vals

        wait_idx(0, sidx0)
        issue_gather(0, sgat0)

        @pl.loop(0, CPW, step=2)
        def _cell_loop(ci):
            c = base + ci

            wait_idx(1, sidx1)
            issue_gather(1, sgat1)
            wait_gather(0, sgat0)

            @pl.when(ci + 2 < CPW)
            def _():
                issue_idx(c + 2, 0, sidx0)

            @pl.when(ci >= 2)
            def _():
                wait_out(0, sout0)

            transpose(0)
            issue_out(c, 0, sout0)

            @pl.when(ci + 2 < CPW)
            def _():
                wait_idx(0, sidx0)
                issue_gather(0, sgat0)

            wait_gather(1, sgat1)

            @pl.when(ci + 3 < CPW)
            def _():
                issue_idx(c + 3, 1, sidx1)

            @pl.when(ci >= 2)
            def _():
                wait_out(1, sout1)

            transpose(1)
            issue_out(c + 1, 1, sout1)

        wait_out(0, sout0)
        wait_out(1, sout1)

    return gather_kernel


def kernel(inputs, emb_matrix):
    idx2 = (inputs.astype(jnp.int32)
            .reshape(128, 128, HIST // 8, 8)
            .transpose(2, 0, 3, 1)
            .reshape(NCELL, CHUNK))
    ep = jnp.pad(emb_matrix, ((0, VPAD - VOCAB), (0, 0)))
    tab4 = ep.reshape(VT, 128, D // 8, 8).transpose(2, 0, 3, 1)
    out5, _, _ = _build()(idx2, tab4)
    return out5.transpose(2, 4, 0, 1, 3).reshape(BATCH, HIST, D)


# speedup vs baseline: 1.0296x; 1.0296x over previous
import functools

import jax
import jax.numpy as jnp
from jax import lax
from jax.experimental import pallas as pl
from jax.experimental.pallas import tpu as pltpu
from jax.experimental.pallas import tpu_sc as plsc

HIST = 200
BATCH = 16384
D = 16
VOCAB = 1000000
NC = 2
NS = 16
NW = NC * NS
CHUNK = 1024
HG = HIST // 8
BT = BATCH // 128
NCELL = HG * BT
CPW = NCELL // NW
VT = 7936
VPAD = VT * 128
K0 = 4
NJ0 = VT // (NW * K0)
MAGIC = 123456.0


@functools.lru_cache(maxsize=None)
def _build():
    mesh = plsc.VectorSubcoreMesh(core_axis_name="c", subcore_axis_name="s")

    @functools.partial(
        pl.kernel,
        out_type=(
            jax.ShapeDtypeStruct((HIST, D // 8, BT, 8, 128), jnp.float32),
            jax.ShapeDtypeStruct((VPAD, D), jnp.float32),
            jax.ShapeDtypeStruct((NC, 16), jnp.float32),
        ),
        mesh=mesh,
        scratch_types=[
            pltpu.VMEM((2, CHUNK), jnp.int32),
            pltpu.VMEM((2, CHUNK, D), jnp.float32),
            pltpu.VMEM((2, 8, D // 8, 8, 128), jnp.float32),
            pltpu.VMEM((2, D // 8, K0, 8, 128), jnp.float32),
            pltpu.VMEM((2, K0 * 128, D), jnp.float32),
            pltpu.VMEM((16,), jnp.float32),
            pltpu.VMEM((16,), jnp.float32),
            pltpu.SemaphoreType.DMA,
            pltpu.SemaphoreType.DMA,
            pltpu.SemaphoreType.DMA,
            pltpu.SemaphoreType.DMA,
            pltpu.SemaphoreType.DMA,
            pltpu.SemaphoreType.DMA,
            pltpu.SemaphoreType.DMA,
            pltpu.SemaphoreType.DMA,
            pltpu.SemaphoreType.DMA,
            pltpu.SemaphoreType.DMA,
            pltpu.SemaphoreType.DMA,
            pltpu.SemaphoreType.DMA,
        ],
        compiler_params=pltpu.CompilerParams(
            use_tc_tiling_on_sc=False, needs_layout_passes=False
        ),
    )
    def gather_kernel(idx_hbm, tab4_hbm, out_hbm, tab_hbm, flags_hbm, idx_v,
                      rows_v, out_v, tin_v, trows_v, flag_v, pflag_v, sidx0,
                      sidx1, sgat0, sgat1, sout0, sout1, stin0, stin1,
                      stout0, stout1, sflag, spoll):
        cid = lax.axis_index("c")
        sid = lax.axis_index("s")
        wid = sid * NC + cid
        base = wid * CPW
        lane = lax.iota(jnp.int32, 16)

        @pl.when(sid == 0)
        def _():
            flag_v[...] = jnp.zeros((16,), jnp.float32)
            pltpu.async_copy(flag_v, flags_hbm.at[cid], sflag).wait()

        def idx_slice(cell):
            return idx_hbm.at[cell, :]

        def issue_idx(cell, b, sem):
            pltpu.async_copy(idx_slice(cell), idx_v.at[b], sem)

        def wait_idx(b, sem):
            pltpu.make_async_copy(idx_slice(0), idx_v.at[b], sem).wait()

        lo0 = wid * (NJ0 * K0)

        def tin_slice(j):
            return tab4_hbm.at[:, pl.ds(lo0 + j * K0, K0), :, :]

        _stin = (stin0, stin1)
        _stout = (stout0, stout1)

        def issue_tin(j, b):
            pltpu.async_copy(tin_slice(j), tin_v.at[b], _stin[b])

        def wait_tin(b):
            pltpu.make_async_copy(tin_slice(0), tin_v.at[b], _stin[b]).wait()

        def trows_slice(j):
            return tab_hbm.at[pl.ds((lo0 + j * K0) * 128, K0 * 128), :]

        def issue_tout(j, b):
            pltpu.async_copy(trows_v.at[b], trows_slice(j), _stout[b])

        def wait_tout(b):
            pltpu.make_async_copy(trows_v.at[b], trows_slice(0),
                                  _stout[b]).wait()

        def transpose_vt(b):
            @plsc.parallel_loop(0, K0 * 8)
            def _vl_loop(i):
                k = i >> 3
                v16 = (i & 7) * 16
                rowv = k * 128 + v16 + lane
                for dg in range(D // 8):
                    for ds in range(8):
                        col = jnp.full((16,), dg * 8 + ds, jnp.int32)
                        vals = tin_v[b, dg, k, ds, pl.ds(v16, 16)]
                        plsc.store_scatter(trows_v.at[b], [rowv, col], vals)

        issue_idx(base, 0, sidx0)
        issue_idx(base + 1, 1, sidx1)

        issue_tin(0, 0)

        @pl.loop(0, NJ0, step=2)
        def _t_loop(j):
            issue_tin(j + 1, 1)
            wait_tin(0)

            @pl.when(j >= 2)
            def _():
                wait_tout(0)

            transpose_vt(0)
            issue_tout(j, 0)

            @pl.when(j + 2 < NJ0)
            def _():
                issue_tin(j + 2, 0)

            wait_tin(1)

            @pl.when(j >= 2)
            def _():
                wait_tout(1)

            transpose_vt(1)
            issue_tout(j + 1, 1)

        wait_tout(0)
        wait_tout(1)
        plsc.subcore_barrier()

        @pl.when(sid == 0)
        def _():
            flag_v[...] = jnp.full((16,), MAGIC, jnp.float32)
            pltpu.async_copy(flag_v, flags_hbm.at[cid], sflag).wait()

            def _poll_body(carry):
                pltpu.async_copy(flags_hbm.at[1 - cid], pflag_v, spoll).wait()
                hits = (pflag_v[...] == MAGIC).astype(jnp.int32)
                return jnp.sum(hits) != 16

            lax.while_loop(lambda c: c, _poll_body, jnp.bool_(True))

        plsc.subcore_barrier()

        def out_slice(cell):
            return out_hbm.at[pl.ds((cell // BT) * 8, 8), :, cell % BT, :, :]

        def issue_gather(b, sem):
            pltpu.async_copy(tab_hbm.at[idx_v.at[b]], rows_v.at[b], sem)

        def wait_gather(b, sem):
            pltpu.make_async_copy(tab_hbm.at[idx_v.at[b]], rows_v.at[b],
                                  sem).wait()

        def issue_out(cell, b, sem):
            pltpu.async_copy(out_v.at[b], out_slice(cell), sem)

        def wait_out(b, sem):
            pltpu.make_async_copy(out_v.at[b], out_slice(0), sem).wait()

        def transpose(b):
            @plsc.parallel_loop(0, 8)
            def _hs_loop(hs):
                row_base = hs * 128
                for dg in range(D // 8):
                    for ds in range(8):
                        col = jnp.full((16,), dg * 8 + ds, jnp.int32)
                        for j in range(8):
                            rows = row_base + j * 16 + lane
                            vals = plsc.load_gather(rows_v.at[b], [rows, col])
                            out_v[b, hs, dg, ds, pl.ds(j * 16, 16)] = vals

        wait_idx(0, sidx0)
        issue_gather(0, sgat0)

        @pl.loop(0, CPW, step=2)
        def _cell_loop(ci):
            c = base + ci

            wait_idx(1, sidx1)
            issue_gather(1, sgat1)
            wait_gather(0, sgat0)

            @pl.when(ci + 2 < CPW)
            def _():
                issue_idx(c + 2, 0, sidx0)

            @pl.when(ci >= 2)
            def _():
                wait_out(0, sout0)

            transpose(0)
            issue_out(c, 0, sout0)

            @pl.when(ci + 2 < CPW)
            def _():
                wait_idx(0, sidx0)
                issue_gather(0, sgat0)

            wait_gather(1, sgat1)

            @pl.when(ci + 3 < CPW)
            def _():
                issue_idx(c + 3, 1, sidx1)

            @pl.when(ci >= 2)
            def _():
                wait_out(1, sout1)

            transpose(1)
            issue_out(c + 1, 1, sout1)

        wait_out(0, sout0)
        wait_out(1, sout1)

    return gather_kernel


def kernel(inputs, emb_matrix):
    idx2 = (inputs.astype(jnp.int32)
            .reshape(128, 128, HIST // 8, 8)
            .transpose(2, 0, 3, 1)
            .reshape(NCELL, CHUNK))
    ep = jnp.pad(emb_matrix, ((0, VPAD - VOCAB), (0, 0)))
    tab4 = ep.reshape(VT, 128, D // 8, 8).transpose(2, 0, 3, 1)
    out5, _, _ = _build()(idx2, tab4)
    return out5.transpose(2, 4, 0, 1, 3).reshape(BATCH, HIST, D)
